# baseline (speedup 1.0000x reference)
import functools

import jax
import jax.numpy as jnp
from jax import lax
from jax.experimental import pallas as pl
from jax.experimental.pallas import tpu as pltpu
from jax.experimental.pallas import tpu_sc as plsc

_DEG = 32
_NW = 32
_K = 3072
_RPW = _K // _NW
_CH = 8
_NCH = _RPW // _CH
_BLK = 384


def _tc_fused_body(self_ref, neigh_ref, wn_ref, ws_ref, out_ref):
    neigh_sum = jnp.sum(neigh_ref[...], axis=1)
    acc = jnp.dot(self_ref[...], ws_ref[...], preferred_element_type=jnp.float32)
    acc = acc + jnp.dot(neigh_sum, wn_ref[...], preferred_element_type=jnp.float32)
    out_ref[...] = jnp.maximum(acc, 0.0)


def _tc_epilogue_body(full_ref, self_ref, sums_ref, wn_ref, ws_ref, out_ref):
    del full_ref
    acc = jnp.dot(self_ref[...], ws_ref[...], preferred_element_type=jnp.float32)
    acc = acc + jnp.dot(sums_ref[...], wn_ref[...], preferred_element_type=jnp.float32)
    out_ref[...] = jnp.maximum(acc, 0.0)


def _sc_body(neigh_hbm, out_hbm, buf0, buf1, acc_v, sem0, sem1):
    wid = lax.axis_index("s") * 2 + lax.axis_index("c")
    base = wid * _RPW

    def reduce_chunk(buf):
        def body(c, carry):
            for j in range(8):
                a = buf[c, 0, pl.ds(j * 16, 16)]
                for k in range(1, _DEG):
                    a = a + buf[c, k, pl.ds(j * 16, 16)]
                acc_v[c, pl.ds(j * 16, 16)] = a
            return carry

        lax.fori_loop(0, _CH, body, 0)

    bufs = (buf0, buf1)
    sems = (sem0, sem1)
    copies = [None, None]
    copies[0] = pltpu.async_copy(neigh_hbm.at[pl.ds(base, _CH)], buf0, sem0)
    for ci in range(_NCH):
        cur = ci % 2
        if ci + 1 < _NCH:
            copies[1 - cur] = pltpu.async_copy(
                neigh_hbm.at[pl.ds(base + (ci + 1) * _CH, _CH)],
                bufs[1 - cur],
                sems[1 - cur],
            )
        copies[cur].wait()
        reduce_chunk(bufs[cur])
        pltpu.sync_copy(acc_v, out_hbm.at[pl.ds(wid * _RPW + ci * _CH, _CH)])


def _sc_neighbor_sums(neigh_vecs, k_rows):
    mesh = plsc.VectorSubcoreMesh(core_axis_name="c", subcore_axis_name="s")
    d_in = neigh_vecs.shape[-1]
    run = pl.kernel(
        _sc_body,
        out_type=jax.ShapeDtypeStruct((k_rows, d_in), jnp.float32),
        mesh=mesh,
        scratch_types=[
            pltpu.VMEM((_CH, _DEG, d_in), jnp.float32),
            pltpu.VMEM((_CH, _DEG, d_in), jnp.float32),
            pltpu.VMEM((_CH, d_in), jnp.float32),
            pltpu.SemaphoreType.DMA,
            pltpu.SemaphoreType.DMA,
        ],
    )
    return run(neigh_vecs)


def kernel(self_vecs, neigh_vecs, neigh_weights, self_weights):
    n, d_in = self_vecs.shape
    deg = neigh_vecs.shape[1]
    d_out = neigh_weights.shape[1]
    n_tail = n - _K
    sc_blocks = _K // _BLK
    wn = neigh_weights * (1.0 / deg)

    sc_sums = _sc_neighbor_sums(neigh_vecs, _K)

    tc_out = pl.pallas_call(
        _tc_fused_body,
        grid=(pl.cdiv(n_tail, _BLK),),
        in_specs=[
            pl.BlockSpec((_BLK, d_in), lambda i, sb=sc_blocks: (sb + i, 0)),
            pl.BlockSpec((_BLK, deg, d_in), lambda i, sb=sc_blocks: (sb + i, 0, 0)),
            pl.BlockSpec((d_in, d_out), lambda i: (0, 0)),
            pl.BlockSpec((d_in, d_out), lambda i: (0, 0)),
        ],
        out_specs=pl.BlockSpec((_BLK, d_out), lambda i, sb=sc_blocks: (sb + i, 0)),
        out_shape=jax.ShapeDtypeStruct((n, d_out), jnp.float32),
        compiler_params=pltpu.CompilerParams(
            dimension_semantics=("arbitrary",),
        ),
    )(self_vecs, neigh_vecs, wn, self_weights)

    out = pl.pallas_call(
        _tc_epilogue_body,
        grid=(sc_blocks,),
        in_specs=[
            pl.BlockSpec(memory_space=pl.ANY),
            pl.BlockSpec((_BLK, d_in), lambda i: (i, 0)),
            pl.BlockSpec((_BLK, d_in), lambda i: (i, 0)),
            pl.BlockSpec((d_in, d_out), lambda i: (0, 0)),
            pl.BlockSpec((d_in, d_out), lambda i: (0, 0)),
        ],
        out_specs=pl.BlockSpec((_BLK, d_out), lambda i: (i, 0)),
        out_shape=jax.ShapeDtypeStruct((n, d_out), jnp.float32),
        input_output_aliases={0: 0},
        compiler_params=pltpu.CompilerParams(
            dimension_semantics=("arbitrary",),
        ),
    )(tc_out, self_vecs, sc_sums, wn, self_weights)
    return out


# speedup vs baseline: 1.4640x; 1.4640x over previous
import jax
import jax.numpy as jnp
from jax.experimental import pallas as pl
from jax.experimental.pallas import tpu as pltpu

_DEG = 32
_BLOCK = 400


def _fused_body(self_ref, neigh_ref, wn_ref, ws_ref, out_ref):
    neigh_sum = jnp.sum(neigh_ref[...], axis=1)
    acc = jnp.dot(self_ref[...], ws_ref[...], preferred_element_type=jnp.float32)
    acc = acc + jnp.dot(neigh_sum, wn_ref[...], preferred_element_type=jnp.float32)
    out_ref[...] = jnp.maximum(acc, 0.0)


def kernel(self_vecs, neigh_vecs, neigh_weights, self_weights):
    n, d_in = self_vecs.shape
    deg = neigh_vecs.shape[1]
    d_out = neigh_weights.shape[1]
    wn = neigh_weights * (1.0 / deg)
    return pl.pallas_call(
        _fused_body,
        grid=(pl.cdiv(n, _BLOCK),),
        in_specs=[
            pl.BlockSpec((_BLOCK, d_in), lambda i: (i, 0)),
            pl.BlockSpec((_BLOCK, deg, d_in), lambda i: (i, 0, 0)),
            pl.BlockSpec((d_in, d_out), lambda i: (0, 0)),
            pl.BlockSpec((d_in, d_out), lambda i: (0, 0)),
        ],
        out_specs=pl.BlockSpec((_BLOCK, d_out), lambda i: (i, 0)),
        out_shape=jax.ShapeDtypeStruct((n, d_out), jnp.float32),
        compiler_params=pltpu.CompilerParams(
            dimension_semantics=("arbitrary",),
        ),
    )(self_vecs, neigh_vecs, wn, self_weights)


# speedup vs baseline: 1.5009x; 1.0253x over previous
import jax
import jax.numpy as jnp
from jax.experimental import pallas as pl
from jax.experimental.pallas import tpu as pltpu

_DEG = 32
_BLOCK = 400


def _fused_body(self_ref, neigh_ref, wn_ref, ws_ref, out_ref):
    neigh_mean = jnp.sum(neigh_ref[...], axis=1) * (1.0 / _DEG)
    acc = jnp.dot(self_ref[...], ws_ref[...], preferred_element_type=jnp.float32)
    acc = acc + jnp.dot(neigh_mean, wn_ref[...], preferred_element_type=jnp.float32)
    out_ref[...] = jnp.maximum(acc, 0.0)


def kernel(self_vecs, neigh_vecs, neigh_weights, self_weights):
    n, d_in = self_vecs.shape
    deg = neigh_vecs.shape[1]
    d_out = neigh_weights.shape[1]
    return pl.pallas_call(
        _fused_body,
        grid=(pl.cdiv(n, _BLOCK),),
        in_specs=[
            pl.BlockSpec((_BLOCK, d_in), lambda i: (i, 0)),
            pl.BlockSpec((_BLOCK, deg, d_in), lambda i: (i, 0, 0)),
            pl.BlockSpec((d_in, d_out), lambda i: (0, 0)),
            pl.BlockSpec((d_in, d_out), lambda i: (0, 0)),
        ],
        out_specs=pl.BlockSpec((_BLOCK, d_out), lambda i: (i, 0)),
        out_shape=jax.ShapeDtypeStruct((n, d_out), jnp.float32),
        compiler_params=pltpu.CompilerParams(
            dimension_semantics=("arbitrary",),
        ),
    )(self_vecs, neigh_vecs, neigh_weights, self_weights)
